# baseline (speedup 1.0000x reference)
import functools

import jax
import jax.numpy as jnp
from jax import lax
from jax.experimental import pallas as pl
from jax.experimental.pallas import tpu as pltpu
from jax.experimental.pallas import tpu_sc as plsc

_DIM = 64
_NUM_WORKERS = 32
_CHUNK = 128


def _build(num_rows: int):
    rows_per_worker = num_rows // _NUM_WORKERS
    n_chunks = rows_per_worker // _CHUNK
    mesh = plsc.VectorSubcoreMesh(core_axis_name="c", subcore_axis_name="s")

    @functools.partial(
        pl.kernel,
        mesh=mesh,
        compiler_params=pltpu.CompilerParams(use_tc_tiling_on_sc=False),
        out_type=jax.ShapeDtypeStruct((num_rows, _DIM), jnp.float32),
        scratch_types=[
            pltpu.VMEM((n_chunks, _CHUNK), jnp.int32),
            pltpu.VMEM((_CHUNK, _DIM), jnp.float32),
            pltpu.SemaphoreType.DMA,
        ],
    )
    def gather_kernel(idx_hbm, table_hbm, out_hbm, idx_v, rows_v, sem):
        wid = lax.axis_index("s") * 2 + lax.axis_index("c")
        base = wid * rows_per_worker
        pltpu.sync_copy(idx_hbm.at[wid], idx_v)

        def body(j, carry):
            pltpu.async_copy(table_hbm.at[idx_v.at[j]], rows_v, sem).wait()
            pltpu.sync_copy(rows_v, out_hbm.at[pl.ds(base + j * _CHUNK, _CHUNK)])
            return carry

        lax.fori_loop(0, n_chunks, body, 0)

    return gather_kernel


def kernel(x, table):
    batch_shape = x.shape
    num_rows = x.size
    idx = x.reshape(_NUM_WORKERS, num_rows // (_NUM_WORKERS * _CHUNK), _CHUNK)
    idx = idx.astype(jnp.int32)
    out = _build(num_rows)(idx, table)
    return out.reshape(*batch_shape, _DIM)


# speedup vs baseline: 1.0456x; 1.0456x over previous
import functools

import jax
import jax.numpy as jnp
from jax import lax
from jax.experimental import pallas as pl
from jax.experimental.pallas import tpu as pltpu
from jax.experimental.pallas import tpu_sc as plsc

_DIM = 64
_NUM_WORKERS = 32
_CHUNK = 128
_MEGA = 5
_ROWS_MEGA = _MEGA * _CHUNK


def _build(num_rows: int):
    rows_per_worker = num_rows // _NUM_WORKERS
    n_chunks = rows_per_worker // _CHUNK
    n_mega = rows_per_worker // _ROWS_MEGA
    n_pairs = n_mega // 2
    mesh = plsc.VectorSubcoreMesh(core_axis_name="c", subcore_axis_name="s")

    @functools.partial(
        pl.kernel,
        mesh=mesh,
        compiler_params=pltpu.CompilerParams(use_tc_tiling_on_sc=False),
        out_type=jax.ShapeDtypeStruct((num_rows, _DIM), jnp.float32),
        scratch_types=[
            pltpu.VMEM((n_chunks, _CHUNK), jnp.int32),
            pltpu.VMEM((_ROWS_MEGA, _DIM), jnp.float32),
            pltpu.VMEM((_ROWS_MEGA, _DIM), jnp.float32),
            pltpu.SemaphoreType.DMA,
            pltpu.SemaphoreType.DMA,
        ],
    )
    def gather_kernel(idx_hbm, table_hbm, out_hbm, idx_v, buf_a, buf_b, sem_a, sem_b):
        wid = lax.axis_index("s") * 2 + lax.axis_index("c")
        base = wid * rows_per_worker
        pltpu.sync_copy(idx_hbm.at[wid], idx_v)

        def start_mega(m, buf, sem):
            for c in range(_MEGA):
                pltpu.async_copy(
                    table_hbm.at[idx_v.at[m * _MEGA + c]],
                    buf.at[pl.ds(c * _CHUNK, _CHUNK)],
                    sem,
                )

        def wait_mega(buf, sem):
            pltpu.make_async_copy(
                out_hbm.at[pl.ds(base, _ROWS_MEGA)], buf, sem
            ).wait()

        def out_mega(m, buf):
            pltpu.sync_copy(
                buf, out_hbm.at[pl.ds(base + m * _ROWS_MEGA, _ROWS_MEGA)]
            )

        start_mega(0, buf_a, sem_a)

        def body(t, carry):
            start_mega(2 * t + 1, buf_b, sem_b)
            wait_mega(buf_a, sem_a)
            out_mega(2 * t, buf_a)
            start_mega(2 * t + 2, buf_a, sem_a)
            wait_mega(buf_b, sem_b)
            out_mega(2 * t + 1, buf_b)
            return carry

        lax.fori_loop(0, n_pairs - 1, body, 0)
        start_mega(n_mega - 1, buf_b, sem_b)
        wait_mega(buf_a, sem_a)
        out_mega(n_mega - 2, buf_a)
        wait_mega(buf_b, sem_b)
        out_mega(n_mega - 1, buf_b)

    return gather_kernel


def kernel(x, table):
    batch_shape = x.shape
    num_rows = x.size
    idx = x.reshape(_NUM_WORKERS, num_rows // (_NUM_WORKERS * _CHUNK), _CHUNK)
    idx = idx.astype(jnp.int32)
    out = _build(num_rows)(idx, table)
    return out.reshape(*batch_shape, _DIM)
